# baseline (speedup 1.0000x reference)
import numpy as np

import jax
import jax.numpy as jnp
from jax import lax
from jax.experimental import pallas as pl
from jax.experimental.pallas import tpu as pltpu

_B, _N, _H, _E = 4, 256, 128, 3
_NE = _N * _E

_kk = np.arange(_NE)[:, None]
_col = np.arange(_NE)[None, :]
_SELCAT = np.asarray(_kk == (_E * (_col % _N) + _col // _N),
                     dtype=jnp.bfloat16)


def _mp_body(adj_ref, ne_ref, er_hbm, mrep_hbm, sel_hbm, w1t_ref, bmsg_ref,
             w2t_ref, wuat_ref, wubt_ref, bupd_ref, out_ref,
             er_s, mrep_s, sel_s, sem_e, sem_m, sem_s):
    f32 = jnp.float32
    cps = [pltpu.make_async_copy(er_hbm.at[b], er_s.at[b], sem_e)
           for b in range(_B)]
    for cp in cps:
        cp.start()
    cpm = pltpu.make_async_copy(mrep_hbm, mrep_s, sem_m)
    cpm.start()
    cpsel = pltpu.make_async_copy(sel_hbm, sel_s, sem_s)
    cpsel.start()

    maskf = (adj_ref[...] > 0).astype(f32)
    ne_all = ne_ref[...].reshape(_B * _N, _H)
    pre_all = (jnp.dot(ne_all, w1t_ref[...], preferred_element_type=f32)
               + bmsg_ref[...])
    pre_cols = jnp.concatenate(
        [pre_all[b * _N:(b + 1) * _N, :] for b in range(_B)], axis=1)
    term1_cols = jnp.dot(maskf, pre_cols,
                         preferred_element_type=f32)

    cpm.wait()
    mrep = mrep_s[...]
    colsums = []
    for b in range(_B):
        cps[b].wait()
        prod = mrep * er_s[b]
        colsums.append(jnp.sum(prod, axis=0, keepdims=True))
    colsum_all = jnp.concatenate(colsums, axis=0)

    cpsel.wait()
    merow_all = jnp.dot(colsum_all.astype(jnp.bfloat16), sel_s[...],
                        preferred_element_type=f32)
    me_all = jnp.concatenate(
        [jnp.concatenate([merow_all[b:b + 1, c * _N:(c + 1) * _N]
                          for c in range(_E)], axis=0)
         for b in range(_B)], axis=1)
    term2_stack = lax.dot_general(
        me_all, w2t_ref[...], (((0,), (0,)), ((), ())),
        preferred_element_type=f32)

    term1_stack = jnp.concatenate(
        [term1_cols[:, b * _H:(b + 1) * _H] for b in range(_B)], axis=0)
    msgs = term1_stack + term2_stack
    h = (jnp.dot(ne_all, wuat_ref[...], preferred_element_type=f32)
         + jnp.dot(msgs, wubt_ref[...], preferred_element_type=f32)
         + bupd_ref[...])
    out_ref[...] = jnp.maximum(h, 0.0).reshape(_B, _N, _H)


@jax.jit
def _run(node_embeddings, edge_relations, adjacency, W_msg, b_msg, W_upd,
         b_upd):
    B, N, H = node_embeddings.shape
    E = edge_relations.shape[-1]
    NE = N * E
    er2 = edge_relations.reshape(B, N, NE)
    maskrep = jnp.repeat(
        (adjacency > 0).astype(jnp.float32).T, E, axis=1)
    W1T = W_msg[:, :H].T
    W2T = W_msg[:, H:].T
    WuAT = W_upd[:, :H].T
    WuBT = W_upd[:, H:].T
    bmsg2 = b_msg.reshape(1, H)
    bupd2 = b_upd.reshape(1, H)
    hbm = pltpu.MemorySpace.HBM
    return pl.pallas_call(
        _mp_body,
        in_specs=[
            pl.BlockSpec((N, N), lambda: (0, 0)),
            pl.BlockSpec((B, N, H), lambda: (0, 0, 0)),
            pl.BlockSpec(memory_space=hbm),
            pl.BlockSpec(memory_space=hbm),
            pl.BlockSpec(memory_space=hbm),
            pl.BlockSpec((H, H), lambda: (0, 0)),
            pl.BlockSpec((1, H), lambda: (0, 0)),
            pl.BlockSpec((E, H), lambda: (0, 0)),
            pl.BlockSpec((H, H), lambda: (0, 0)),
            pl.BlockSpec((H, H), lambda: (0, 0)),
            pl.BlockSpec((1, H), lambda: (0, 0)),
        ],
        out_specs=pl.BlockSpec((B, N, H), lambda: (0, 0, 0)),
        out_shape=jax.ShapeDtypeStruct((B, N, H), jnp.float32),
        scratch_shapes=[
            pltpu.VMEM((B, N, NE), jnp.float32),
            pltpu.VMEM((N, NE), jnp.float32),
            pltpu.VMEM((NE, NE), jnp.bfloat16),
            pltpu.SemaphoreType.DMA,
            pltpu.SemaphoreType.DMA,
            pltpu.SemaphoreType.DMA,
        ],
    )(adjacency, node_embeddings, er2, maskrep, jnp.asarray(_SELCAT), W1T,
      bmsg2, W2T, WuAT, WuBT, bupd2)


def kernel(node_embeddings, edge_relations, adjacency, W_msg, b_msg, W_upd,
           b_upd):
    return _run(node_embeddings, edge_relations, adjacency, W_msg, b_msg,
                W_upd, b_upd)
